# baseline (speedup 1.0000x reference)
import functools

import jax
import jax.numpy as jnp
from jax import lax
from jax.experimental import pallas as pl
from jax.experimental.pallas import tpu as pltpu

N = 10000
D_IN = 512
H = 2
C = 256


def _ln(h, g, b, eps=1e-5):
    mu = jnp.mean(h, axis=-1, keepdims=True)
    var = jnp.mean((h - mu) ** 2, axis=-1, keepdims=True)
    return (h - mu) * lax.rsqrt(var + eps) * g + b


def _mlp_body(hg_ref, Wa_ref, W1_ref, W2_ref, W3_ref, ba_ref, ga_ref, bta_ref,
              b1_ref, g1_ref, bt1_ref, b2_ref, g2_ref, bt2_ref, b3_ref, y_ref):
    h = jnp.maximum(hg_ref[...], 0.0)
    h = jax.lax.dot_general(h, Wa_ref[...], (((1,), (0,)), ((), ())),
                            preferred_element_type=jnp.float32) + ba_ref[...]
    h = _ln(h, ga_ref[...], bta_ref[...])
    h = jnp.maximum(h, 0.0)
    h = jax.lax.dot_general(h, W1_ref[...], (((1,), (0,)), ((), ())),
                            preferred_element_type=jnp.float32) + b1_ref[...]
    h = _ln(h, g1_ref[...], bt1_ref[...])
    h = jnp.tanh(jnp.maximum(h, 0.0))
    h = jax.lax.dot_general(h, W2_ref[...], (((1,), (0,)), ((), ())),
                            preferred_element_type=jnp.float32) + b2_ref[...]
    h = _ln(h, g2_ref[...], bt2_ref[...])
    h = jnp.maximum(h, 0.0)
    y = jax.lax.dot_general(h, W3_ref[...], (((1,), (0,)), ((), ())),
                            preferred_element_type=jnp.float32) + b3_ref[...]
    y_ref[...] = y


def _mlp_head(h_gat, Wa, ba, ga, bta, W1, b1, g1, bt1, W2, b2, g2, bt2, W3, b3):
    bn = 400
    W3p = jnp.zeros((64, 128), jnp.float32).at[:, :3].set(W3)
    b3p = jnp.zeros((1, 128), jnp.float32).at[0, :3].set(b3)
    row = lambda v: v.reshape(1, -1)
    grid = N // bn
    y = pl.pallas_call(
        _mlp_body,
        grid=(grid,),
        in_specs=[
            pl.BlockSpec((bn, D_IN), lambda i: (i, 0)),
            pl.BlockSpec((D_IN, 256), lambda i: (0, 0)),
            pl.BlockSpec((256, 128), lambda i: (0, 0)),
            pl.BlockSpec((128, 64), lambda i: (0, 0)),
            pl.BlockSpec((64, 128), lambda i: (0, 0)),
            pl.BlockSpec((1, 256), lambda i: (0, 0)),
            pl.BlockSpec((1, 256), lambda i: (0, 0)),
            pl.BlockSpec((1, 256), lambda i: (0, 0)),
            pl.BlockSpec((1, 128), lambda i: (0, 0)),
            pl.BlockSpec((1, 128), lambda i: (0, 0)),
            pl.BlockSpec((1, 128), lambda i: (0, 0)),
            pl.BlockSpec((1, 64), lambda i: (0, 0)),
            pl.BlockSpec((1, 64), lambda i: (0, 0)),
            pl.BlockSpec((1, 64), lambda i: (0, 0)),
            pl.BlockSpec((1, 128), lambda i: (0, 0)),
        ],
        out_specs=pl.BlockSpec((bn, 128), lambda i: (i, 0)),
        out_shape=jax.ShapeDtypeStruct((N, 128), jnp.float32),
    )(h_gat, Wa, W1, W2, W3p, row(ba), row(ga), row(bta), row(b1), row(g1),
      row(bt1), row(b2), row(g2), row(bt2), b3p)
    return y


def _cdist_body(yi_ref, yj_ref, o_ref):
    yi = yi_ref[...]
    yj = yj_ref[...]
    si = jnp.sum(yi * yi, axis=1, keepdims=True)
    sj = jnp.sum(yj * yj, axis=1, keepdims=True)
    dot = jax.lax.dot_general(yi, yj, (((1,), (1,)), ((), ())),
                              preferred_element_type=jnp.float32)
    d2 = si + jnp.transpose(sj) - 2.0 * dot
    d2 = jnp.maximum(d2, 0.0)
    safe = jnp.where(d2 > 0.0, d2, 1.0)
    o_ref[...] = jnp.where(d2 > 0.0, jnp.sqrt(safe), 0.0)


def _cdist(y_pad):
    bm = 400
    return pl.pallas_call(
        _cdist_body,
        grid=(N // bm,),
        in_specs=[
            pl.BlockSpec((bm, 128), lambda i: (i, 0)),
            pl.BlockSpec((N, 128), lambda i: (0, 0)),
        ],
        out_specs=pl.BlockSpec((bm, N), lambda i: (i, 0)),
        out_shape=jax.ShapeDtypeStruct((N, N), jnp.float32),
    )(y_pad, y_pad)


def _gat_jax(x, edge_index, W_conv, att_src, att_dst, b_conv):
    n = x.shape[0]
    loops = jnp.arange(n, dtype=edge_index.dtype)
    src = jnp.concatenate([edge_index[0], loops])
    dst = jnp.concatenate([edge_index[1], loops])
    xp = (x @ W_conv).reshape(n, H, C)
    a_src = jnp.sum(xp * att_src, axis=-1)
    a_dst = jnp.sum(xp * att_dst, axis=-1)
    alpha = a_src[src] + a_dst[dst]
    alpha = jax.nn.leaky_relu(alpha, negative_slope=0.2)
    amax = jax.ops.segment_max(alpha, dst, num_segments=n)
    alpha = jnp.exp(alpha - amax[dst])
    denom = jax.ops.segment_sum(alpha, dst, num_segments=n)
    alpha = alpha / (denom[dst] + 1e-16)
    msg = xp[src] * alpha[:, :, None]
    out = jax.ops.segment_sum(msg, dst, num_segments=n)
    return out.reshape(n, H * C) + b_conv


def kernel(x, edge_index, W_conv, att_src, att_dst, b_conv, Wa, ba, ga, bta,
           W1, b1, g1, bt1, W2, b2, g2, bt2, W3, b3):
    h_gat = _gat_jax(x, edge_index, W_conv, att_src, att_dst, b_conv)
    y = _mlp_head(h_gat, Wa, ba, ga, bta, W1, b1, g1, bt1, W2, b2, g2, bt2,
                  W3, b3)
    return _cdist(y)


# speedup vs baseline: 7.5975x; 7.5975x over previous
import functools

import jax
import jax.numpy as jnp
from jax import lax
from jax.experimental import pallas as pl
from jax.experimental.pallas import tpu as pltpu
from jax.experimental.pallas import tpu_sc as plsc

N = 10000
E = 160000
D_IN = 512
H = 2
C = 256

NP = 10240
NRG = 160
NRANGES = NP // NRG
NW = 32
EPT = E // NW
BLK = 2000
NBLK = E // BLK
NLAST = N - (N // NRG) * NRG

_sc_mesh = plsc.VectorSubcoreMesh(core_axis_name="c", subcore_axis_name="s")
_IOTA16 = lambda: lax.iota(jnp.int32, 16)


def _lane_f(v, l):
    return jnp.sum(jnp.where(_IOTA16() == l, v, jnp.zeros_like(v)))


def _lane_i(v, l):
    return jnp.sum(jnp.where(_IOTA16() == l, v, jnp.zeros_like(v)))


def _stage_a_body(x_ref, wc_ref, ap_ref, xp_ref, a4_ref, ws_ref):
    xp = jax.lax.dot_general(x_ref[...], wc_ref[...], (((1,), (0,)), ((), ())),
                             preferred_element_type=jnp.float32)
    xp_ref[...] = xp
    a4 = jax.lax.dot_general(xp, ap_ref[...], (((1,), (0,)), ((), ())),
                             preferred_element_type=jnp.float32)
    a4_ref[...] = a4[:, :4]
    s = a4[:, 0:2] + a4[:, 2:4]
    s = jnp.where(s >= 0.0, s, 0.2 * s)
    ws_ref[...] = jnp.exp(s)


def _stage_a(x, W_conv, Apad):
    bn = 400
    return pl.pallas_call(
        _stage_a_body,
        grid=(N // bn,),
        in_specs=[
            pl.BlockSpec((bn, D_IN), lambda i: (i, 0)),
            pl.BlockSpec((D_IN, D_IN), lambda i: (0, 0)),
            pl.BlockSpec((D_IN, 128), lambda i: (0, 0)),
        ],
        out_specs=[
            pl.BlockSpec((bn, D_IN), lambda i: (i, 0)),
            pl.BlockSpec((bn, 4), lambda i: (i, 0)),
            pl.BlockSpec((bn, 2), lambda i: (i, 0)),
        ],
        out_shape=[
            jax.ShapeDtypeStruct((N, D_IN), jnp.float32),
            jax.ShapeDtypeStruct((N, 4), jnp.float32),
            jax.ShapeDtypeStruct((N, 2), jnp.float32),
        ],
    )(x, W_conv, Apad)


@functools.partial(
    pl.kernel,
    out_type=[
        jax.ShapeDtypeStruct((E,), jnp.float32),
        jax.ShapeDtypeStruct((E,), jnp.float32),
        jax.ShapeDtypeStruct((NW * 2 * NP,), jnp.float32),
    ],
    mesh=_sc_mesh,
    scratch_types=[
        pltpu.VMEM((4 * NP,), jnp.float32),
        pltpu.VMEM((2 * NP,), jnp.float32),
        pltpu.VMEM((EPT + 16,), jnp.int32),
        pltpu.VMEM((EPT + 16,), jnp.int32),
        pltpu.VMEM((EPT + 16,), jnp.float32),
        pltpu.VMEM((EPT + 16,), jnp.float32),
    ],
    compiler_params=pltpu.CompilerParams(needs_layout_passes=False),
)
def _stage_b(src_hbm, dst_hbm, aflat_hbm, w0_hbm, w1_hbm, denp_hbm,
             a_loc, den_loc, srcb, dstb, w0b, w1b):
    wid = lax.axis_index("s") * 2 + lax.axis_index("c")
    base = wid * EPT
    pltpu.sync_copy(aflat_hbm, a_loc)

    def zero_body(i, _):
        den_loc[pl.ds(16 * i, 16)] = jnp.zeros((16,), jnp.float32)
        return 0

    lax.fori_loop(0, 2 * NP // 16, zero_body, 0)

    pltpu.sync_copy(src_hbm.at[pl.ds(base, EPT)], srcb.at[pl.ds(0, EPT)])
    pltpu.sync_copy(dst_hbm.at[pl.ds(base, EPT)], dstb.at[pl.ds(0, EPT)])

    nchunk = (EPT + 15) // 16

    def edge_body(i, _):
        off = 16 * i
        m = off + _IOTA16() < EPT
        s16 = jnp.where(m, srcb[pl.ds(off, 16)], 0)
        d16 = jnp.where(m, dstb[pl.ds(off, 16)], 0)
        as0 = plsc.load_gather(a_loc, [s16])
        as1 = plsc.load_gather(a_loc, [s16 + NP])
        ad0 = plsc.load_gather(a_loc, [d16 + 2 * NP])
        ad1 = plsc.load_gather(a_loc, [d16 + 3 * NP])
        al0 = as0 + ad0
        al0 = jnp.where(al0 >= 0.0, al0, 0.2 * al0)
        w0v = jnp.exp(al0)
        al1 = as1 + ad1
        al1 = jnp.where(al1 >= 0.0, al1, 0.2 * al1)
        w1v = jnp.exp(al1)
        w0b[pl.ds(off, 16)] = w0v
        w1b[pl.ds(off, 16)] = w1v
        plsc.addupdate_scatter(den_loc, [d16], w0v, mask=m)
        plsc.addupdate_scatter(den_loc, [d16 + NP], w1v, mask=m)
        return 0

    lax.fori_loop(0, nchunk, edge_body, 0)

    pltpu.sync_copy(w0b.at[pl.ds(0, EPT)], w0_hbm.at[pl.ds(base, EPT)])
    pltpu.sync_copy(w1b.at[pl.ds(0, EPT)], w1_hbm.at[pl.ds(base, EPT)])
    pltpu.sync_copy(den_loc, denp_hbm.at[pl.ds(wid * 2 * NP, 2 * NP)])


@functools.partial(
    pl.kernel,
    out_type=[
        jax.ShapeDtypeStruct((N, D_IN), jnp.float32),
        jax.ShapeDtypeStruct((2 * NP,), jnp.float32),
    ],
    mesh=_sc_mesh,
    scratch_types=[
        pltpu.VMEM((NRG, D_IN), jnp.float32),
        pltpu.VMEM((BLK,), jnp.int32),
        pltpu.VMEM((BLK,), jnp.int32),
        pltpu.VMEM((BLK,), jnp.float32),
        pltpu.VMEM((BLK,), jnp.float32),
        pltpu.VMEM((BLK + 16,), jnp.int32),
        pltpu.VMEM((BLK + 16,), jnp.int32),
        pltpu.VMEM((BLK + 16,), jnp.float32),
        pltpu.VMEM((BLK + 16,), jnp.float32),
        pltpu.VMEM((16, D_IN), jnp.float32),
        pltpu.VMEM((16,), jnp.int32),
        pltpu.VMEM((NRG,), jnp.float32),
        pltpu.VMEM((NRG,), jnp.float32),
        pltpu.VMEM((NRG,), jnp.float32),
        pltpu.SemaphoreType.DMA,
    ],
    compiler_params=pltpu.CompilerParams(needs_layout_passes=False),
)
def _stage_c(src_hbm, dst_hbm, w0_hbm, w1_hbm, denp_hbm, ws_hbm, xp_hbm,
             agg_hbm, dent_hbm,
             acc, srcb, dstb, w0b, w1b, q_src, q_ld, q_w0, q_w1, rows,
             idxv, den0v, den1v, tmpv, sem):
    wid = lax.axis_index("s") * 2 + lax.axis_index("c")

    for p in range(2):
        k = 2 * wid + p
        b = NRG * k

        for j in range(NRG // 16):
            den0v[pl.ds(16 * j, 16)] = jnp.zeros((16,), jnp.float32)
            den1v[pl.ds(16 * j, 16)] = jnp.zeros((16,), jnp.float32)

        def den_body(w, _):
            pltpu.sync_copy(denp_hbm.at[pl.ds(w * 2 * NP + b, NRG)], tmpv)
            for j in range(NRG // 16):
                sl = pl.ds(16 * j, 16)
                den0v[sl] = den0v[sl] + tmpv[sl]
            pltpu.sync_copy(denp_hbm.at[pl.ds(w * 2 * NP + NP + b, NRG)], tmpv)
            for j in range(NRG // 16):
                sl = pl.ds(16 * j, 16)
                den1v[sl] = den1v[sl] + tmpv[sl]
            return 0

        lax.fori_loop(0, NW, den_body, 0)
        pltpu.sync_copy(ws_hbm.at[pl.ds(b, NRG)], tmpv)
        for j in range(NRG // 16):
            sl = pl.ds(16 * j, 16)
            den0v[sl] = den0v[sl] + tmpv[sl] + 1e-16
        pltpu.sync_copy(ws_hbm.at[pl.ds(NP + b, NRG)], tmpv)
        for j in range(NRG // 16):
            sl = pl.ds(16 * j, 16)
            den1v[sl] = den1v[sl] + tmpv[sl] + 1e-16

        def acc_zero(r, _):
            for cc in range(D_IN // 16):
                acc[r, pl.ds(16 * cc, 16)] = jnp.zeros((16,), jnp.float32)
            return 0

        lax.fori_loop(0, NRG, acc_zero, 0)

        def blk_body(blk, _):
            eb = BLK * blk
            pltpu.sync_copy(src_hbm.at[pl.ds(eb, BLK)], srcb)
            pltpu.sync_copy(dst_hbm.at[pl.ds(eb, BLK)], dstb)
            pltpu.sync_copy(w0_hbm.at[pl.ds(eb, BLK)], w0b)
            pltpu.sync_copy(w1_hbm.at[pl.ds(eb, BLK)], w1b)

            def scan_body(i, qn):
                off = 16 * i
                d16 = dstb[pl.ds(off, 16)]
                m = (d16 >= b) & (d16 < b + NRG)
                plsc.store_compressed(q_src.at[pl.ds(qn, 16)],
                                      srcb[pl.ds(off, 16)], mask=m)
                plsc.store_compressed(q_ld.at[pl.ds(qn, 16)], d16 - b, mask=m)
                plsc.store_compressed(q_w0.at[pl.ds(qn, 16)],
                                      w0b[pl.ds(off, 16)], mask=m)
                plsc.store_compressed(q_w1.at[pl.ds(qn, 16)],
                                      w1b[pl.ds(off, 16)], mask=m)
                return qn + jnp.sum(m.astype(jnp.int32))

            qn = lax.fori_loop(0, BLK // 16, scan_body, jnp.int32(0))

            q_src[pl.ds(qn, 16)] = jnp.zeros((16,), jnp.int32)
            q_ld[pl.ds(qn, 16)] = jnp.zeros((16,), jnp.int32)
            q_w0[pl.ds(qn, 16)] = jnp.zeros((16,), jnp.float32)
            q_w1[pl.ds(qn, 16)] = jnp.zeros((16,), jnp.float32)
            nq = (qn + 15) // 16

            def q_body(qi, _):
                off = 16 * qi
                idxv[...] = q_src[pl.ds(off, 16)]
                pltpu.async_copy(xp_hbm.at[idxv], rows, sem).wait()
                ld16 = q_ld[pl.ds(off, 16)]
                a0 = q_w0[pl.ds(off, 16)] / plsc.load_gather(den0v, [ld16])
                a1 = q_w1[pl.ds(off, 16)] / plsc.load_gather(den1v, [ld16])

                def l_body(l, _):
                    r = _lane_i(ld16, l)
                    c0 = _lane_f(a0, l)
                    c1 = _lane_f(a1, l)
                    for cc in range(D_IN // 16):
                        coef = c0 if cc < C // 16 else c1
                        sl = pl.ds(16 * cc, 16)
                        acc[r, sl] = acc[r, sl] + coef * rows[l, sl]
                    return 0

                lax.fori_loop(0, 16, l_body, 0)
                return 0

            lax.fori_loop(0, nq, q_body, 0)
            return 0

        lax.fori_loop(0, NBLK, blk_body, 0)

        @pl.when(b + NRG <= N)
        def _():
            pltpu.sync_copy(acc, agg_hbm.at[pl.ds(b, NRG)])
            pltpu.sync_copy(den0v, dent_hbm.at[pl.ds(b, NRG)])
            pltpu.sync_copy(den1v, dent_hbm.at[pl.ds(NP + b, NRG)])

        @pl.when((b < N) & (b + NRG > N))
        def _():
            pltpu.sync_copy(acc.at[pl.ds(0, NLAST)], agg_hbm.at[pl.ds(b, NLAST)])
            pltpu.sync_copy(den0v.at[pl.ds(0, NLAST)],
                            dent_hbm.at[pl.ds(b, NLAST)])
            pltpu.sync_copy(den1v.at[pl.ds(0, NLAST)],
                            dent_hbm.at[pl.ds(NP + b, NLAST)])


def _ln(h, g, bb, eps=1e-5):
    mu = jnp.mean(h, axis=-1, keepdims=True)
    var = jnp.mean((h - mu) ** 2, axis=-1, keepdims=True)
    return (h - mu) * lax.rsqrt(var + eps) * g + bb


def _mlp_body(agg_ref, xp_ref, ws_ref, dent_ref, bc_ref,
              Wa_ref, W1_ref, W2_ref, W3_ref, ba_ref, ga_ref, bta_ref,
              b1_ref, g1_ref, bt1_ref, b2_ref, g2_ref, bt2_ref, b3_ref, y_ref):
    coef = ws_ref[...] / dent_ref[...]
    xp = xp_ref[...]
    self0 = xp[:, :C] * coef[:, 0:1]
    self1 = xp[:, C:] * coef[:, 1:2]
    h = agg_ref[...] + jnp.concatenate([self0, self1], axis=1) + bc_ref[...]
    h = jnp.maximum(h, 0.0)
    h = jax.lax.dot_general(h, Wa_ref[...], (((1,), (0,)), ((), ())),
                            preferred_element_type=jnp.float32) + ba_ref[...]
    h = _ln(h, ga_ref[...], bta_ref[...])
    h = jnp.maximum(h, 0.0)
    h = jax.lax.dot_general(h, W1_ref[...], (((1,), (0,)), ((), ())),
                            preferred_element_type=jnp.float32) + b1_ref[...]
    h = _ln(h, g1_ref[...], bt1_ref[...])
    h = jnp.tanh(jnp.maximum(h, 0.0))
    h = jax.lax.dot_general(h, W2_ref[...], (((1,), (0,)), ((), ())),
                            preferred_element_type=jnp.float32) + b2_ref[...]
    h = _ln(h, g2_ref[...], bt2_ref[...])
    h = jnp.maximum(h, 0.0)
    y = jax.lax.dot_general(h, W3_ref[...], (((1,), (0,)), ((), ())),
                            preferred_element_type=jnp.float32) + b3_ref[...]
    y_ref[...] = y


def _mlp_head(agg, xp, ws, dentT, b_conv, Wa, ba, ga, bta, W1, b1, g1, bt1,
              W2, b2, g2, bt2, W3, b3):
    bn = 400
    W3p = jnp.zeros((64, 128), jnp.float32).at[:, :3].set(W3)
    b3p = jnp.zeros((1, 128), jnp.float32).at[0, :3].set(b3)
    row = lambda v: v.reshape(1, -1)
    y = pl.pallas_call(
        _mlp_body,
        grid=(N // bn,),
        in_specs=[
            pl.BlockSpec((bn, D_IN), lambda i: (i, 0)),
            pl.BlockSpec((bn, D_IN), lambda i: (i, 0)),
            pl.BlockSpec((bn, 2), lambda i: (i, 0)),
            pl.BlockSpec((bn, 2), lambda i: (i, 0)),
            pl.BlockSpec((1, D_IN), lambda i: (0, 0)),
            pl.BlockSpec((D_IN, 256), lambda i: (0, 0)),
            pl.BlockSpec((256, 128), lambda i: (0, 0)),
            pl.BlockSpec((128, 64), lambda i: (0, 0)),
            pl.BlockSpec((64, 128), lambda i: (0, 0)),
            pl.BlockSpec((1, 256), lambda i: (0, 0)),
            pl.BlockSpec((1, 256), lambda i: (0, 0)),
            pl.BlockSpec((1, 256), lambda i: (0, 0)),
            pl.BlockSpec((1, 128), lambda i: (0, 0)),
            pl.BlockSpec((1, 128), lambda i: (0, 0)),
            pl.BlockSpec((1, 128), lambda i: (0, 0)),
            pl.BlockSpec((1, 64), lambda i: (0, 0)),
            pl.BlockSpec((1, 64), lambda i: (0, 0)),
            pl.BlockSpec((1, 64), lambda i: (0, 0)),
            pl.BlockSpec((1, 128), lambda i: (0, 0)),
        ],
        out_specs=pl.BlockSpec((bn, 128), lambda i: (i, 0)),
        out_shape=jax.ShapeDtypeStruct((N, 128), jnp.float32),
    )(agg, xp, ws, dentT, b_conv.reshape(1, -1), Wa, W1, W2, W3p, row(ba),
      row(ga), row(bta), row(b1), row(g1), row(bt1), row(b2), row(g2),
      row(bt2), b3p)
    return y


def _cdist_body(yi_ref, yj_ref, o_ref):
    yi = yi_ref[...]
    yj = yj_ref[...]
    si = jnp.sum(yi * yi, axis=1, keepdims=True)
    sj = jnp.sum(yj * yj, axis=1, keepdims=True)
    dot = jax.lax.dot_general(yi, yj, (((1,), (1,)), ((), ())),
                              preferred_element_type=jnp.float32)
    d2 = si + jnp.transpose(sj) - 2.0 * dot
    d2 = jnp.maximum(d2, 0.0)
    safe = jnp.where(d2 > 0.0, d2, 1.0)
    o_ref[...] = jnp.where(d2 > 0.0, jnp.sqrt(safe), 0.0)


def _cdist(y_pad):
    bm = 400
    return pl.pallas_call(
        _cdist_body,
        grid=(N // bm,),
        in_specs=[
            pl.BlockSpec((bm, 128), lambda i: (i, 0)),
            pl.BlockSpec((N, 128), lambda i: (0, 0)),
        ],
        out_specs=pl.BlockSpec((bm, N), lambda i: (i, 0)),
        out_shape=jax.ShapeDtypeStruct((N, N), jnp.float32),
    )(y_pad, y_pad)


def kernel(x, edge_index, W_conv, att_src, att_dst, b_conv, Wa, ba, ga, bta,
           W1, b1, g1, bt1, W2, b2, g2, bt2, W3, b3):
    A4 = jnp.zeros((D_IN, 128), jnp.float32)
    A4 = A4.at[:C, 0].set(att_src[0, 0])
    A4 = A4.at[C:, 1].set(att_src[0, 1])
    A4 = A4.at[:C, 2].set(att_dst[0, 0])
    A4 = A4.at[C:, 3].set(att_dst[0, 1])

    xp, a4, ws = _stage_a(x, W_conv, A4)

    a_flat = jnp.zeros((4, NP), jnp.float32).at[:, :N].set(a4.T).reshape(-1)
    ws2 = jnp.zeros((2, NP), jnp.float32).at[:, :N].set(ws.T).reshape(-1)

    src = edge_index[0]
    dst = edge_index[1]
    w0, w1, denp = _stage_b(src, dst, a_flat)
    agg, dent = _stage_c(src, dst, w0, w1, denp, ws2, xp)

    dentT = dent.reshape(2, NP)[:, :N].T
    y = _mlp_head(agg, xp, ws, dentT, b_conv, Wa, ba, ga, bta, W1, b1, g1,
                  bt1, W2, b2, g2, bt2, W3, b3)
    return _cdist(y)
